# Initial kernel scaffold; baseline (speedup 1.0000x reference)
import jax
import jax.numpy as jnp
from jax.experimental import pallas as pl


def kernel(ui_graph, iu_graph, mm_ui_graph_0, mm_ui_graph_1, mm_iu_graph_0, mm_iu_graph_1, mm_feats_0, mm_feats_1, enc0_W1, enc0_b1, enc0_W2, enc0_b2, enc1_W1, enc1_b1, enc1_W2, enc1_b2, user_emb, item_emb, w_q, w_k, w_cat):
    raise NotImplementedError("write your pallas kernel here")


# speedup vs baseline: 1.7662x; 1.7662x over previous
import jax
import jax.numpy as jnp
from jax.experimental import pallas as pl
from jax.experimental.pallas import tpu as pltpu

_N = 4096
_D = 64
_BM = 512
_BM_ID = 256


def _l2n(x):
    n = jnp.sqrt(jnp.sum(x * x, axis=1, keepdims=True))
    return x / jnp.maximum(n, 1e-12)


def _lrelu(x):
    return jnp.where(x >= 0, x, 0.01 * x)


def _dot(a, b):
    return jnp.dot(a, b, preferred_element_type=jnp.float32)


def _row_spec(bm, w):
    return pl.BlockSpec((bm, w), lambda i: (i, 0))


def _full_spec(h, w):
    return pl.BlockSpec((h, w), lambda i: (0, 0))


_PARAMS = pltpu.CompilerParams(dimension_semantics=("arbitrary",))


def _enc_body(f0, f1, w10, b10, w20, b20, w11, b11, w21, b21, o0, o1):
    h0 = _lrelu(_dot(f0[...], w10[...]) + b10[...])
    o0[...] = _lrelu(_dot(h0, w20[...]) + b20[...])
    h1 = _lrelu(_dot(f1[...], w11[...]) + b11[...])
    o1[...] = _lrelu(_dot(h1, w21[...]) + b21[...])


def _id_body(ui0, ui1, iu0, iu1, iemb, uemb, wcat, ue, ie, ou, oi):
    wc = wcat[...]
    ws = wc[0:64] + wc[64:128] + wc[128:192] + wc[192:256]
    eu = _dot(iemb[...], ws) * 0.5
    ei = _dot(uemb[...], ws) * 0.5
    um = _dot(ui0[...] + ui1[...], eu)
    im = _dot(iu0[...] + iu1[...], ei)
    ou[...] = ue[...] + 0.36 * _l2n(um)
    oi[...] = ie[...] + 0.36 * _l2n(im)


def _passA_body(g, rhs, out):
    out[...] = _dot(g[...], rhs[...])


def _passB_body(g, rhs, out, sm):
    t = _dot(g[...], rhs[...])
    out[...] = t
    s = t[:, 128:192]
    s = s - jnp.max(s, axis=1, keepdims=True)
    e = jnp.exp(s)
    sm[...] = e / jnp.sum(e, axis=1, keepdims=True)


def _passC_body(g, rhs, ug0, u1, out, uf):
    t = _dot(g[...], rhs[...])
    out[...] = t
    uf[...] = (ug0[...] + u1[...] + t[:, 128:192]) / 3.0 + 0.02 * (
        _l2n(t[:, 0:64]) + _l2n(t[:, 64:128]))


def _passD_body(g, rhs, ig0, i1, of):
    t = _dot(g[...], rhs[...])
    of[...] = (ig0[...] + i1[...] + t[:, 128:192]) / 3.0 + 0.02 * (
        _l2n(t[:, 0:64]) + _l2n(t[:, 64:128]))


def kernel(ui_graph, iu_graph, mm_ui_graph_0, mm_ui_graph_1, mm_iu_graph_0,
           mm_iu_graph_1, mm_feats_0, mm_feats_1,
           enc0_W1, enc0_b1, enc0_W2, enc0_b2,
           enc1_W1, enc1_b1, enc1_W2, enc1_b2,
           user_emb, item_emb, w_q, w_k, w_cat):
    del w_q, w_k
    f32 = jnp.float32
    n_blk = _N // _BM
    k1 = enc0_W1.shape[1]
    k2 = enc1_W1.shape[0]
    k3 = enc1_W1.shape[1]

    if0, if1 = pl.pallas_call(
        _enc_body,
        grid=(n_blk,),
        in_specs=[
            _row_spec(_BM, _N),
            _row_spec(_BM, k2),
            _full_spec(_N, k1), _full_spec(1, k1),
            _full_spec(k1, _D), _full_spec(1, _D),
            _full_spec(k2, k3), _full_spec(1, k3),
            _full_spec(k3, _D), _full_spec(1, _D),
        ],
        out_specs=[_row_spec(_BM, _D), _row_spec(_BM, _D)],
        out_shape=[jax.ShapeDtypeStruct((_N, _D), f32)] * 2,
        compiler_params=_PARAMS,
    )(mm_feats_0, mm_feats_1,
      enc0_W1, enc0_b1.reshape(1, -1), enc0_W2, enc0_b2.reshape(1, -1),
      enc1_W1, enc1_b1.reshape(1, -1), enc1_W2, enc1_b2.reshape(1, -1))

    n_blk_id = _N // _BM_ID
    ug0, ig0 = pl.pallas_call(
        _id_body,
        grid=(n_blk_id,),
        in_specs=[
            _row_spec(_BM_ID, _N), _row_spec(_BM_ID, _N),
            _row_spec(_BM_ID, _N), _row_spec(_BM_ID, _N),
            _full_spec(_N, _D), _full_spec(_N, _D),
            _full_spec(4 * _D, _D),
            _row_spec(_BM_ID, _D), _row_spec(_BM_ID, _D),
        ],
        out_specs=[_row_spec(_BM_ID, _D), _row_spec(_BM_ID, _D)],
        out_shape=[jax.ShapeDtypeStruct((_N, _D), f32)] * 2,
        compiler_params=_PARAMS,
    )(mm_ui_graph_0, mm_ui_graph_1, mm_iu_graph_0, mm_iu_graph_1,
      item_emb, user_emb, w_cat, user_emb, item_emb)

    w = 3 * _D

    rhs_a = jnp.concatenate([if0, if1, ig0], axis=1)
    out_a = pl.pallas_call(
        _passA_body,
        grid=(n_blk,),
        in_specs=[_row_spec(_BM, _N), _full_spec(_N, w)],
        out_specs=_row_spec(_BM, w),
        out_shape=jax.ShapeDtypeStruct((_N, w), f32),
        compiler_params=_PARAMS,
    )(ui_graph, rhs_a)

    out_b, sm_i1 = pl.pallas_call(
        _passB_body,
        grid=(n_blk,),
        in_specs=[_row_spec(_BM, _N), _full_spec(_N, w)],
        out_specs=[_row_spec(_BM, w), _row_spec(_BM, _D)],
        out_shape=[jax.ShapeDtypeStruct((_N, w), f32),
                   jax.ShapeDtypeStruct((_N, _D), f32)],
        compiler_params=_PARAMS,
    )(iu_graph, out_a)

    rhs_c = jnp.concatenate([out_b[:, 0:128], sm_i1], axis=1)
    u1 = out_a[:, 128:192]
    out_c, u_final = pl.pallas_call(
        _passC_body,
        grid=(n_blk,),
        in_specs=[_row_spec(_BM, _N), _full_spec(_N, w),
                  _row_spec(_BM, _D), _row_spec(_BM, _D)],
        out_specs=[_row_spec(_BM, w), _row_spec(_BM, _D)],
        out_shape=[jax.ShapeDtypeStruct((_N, w), f32),
                   jax.ShapeDtypeStruct((_N, _D), f32)],
        compiler_params=_PARAMS,
    )(ui_graph, rhs_c, ug0, u1)

    i1 = out_b[:, 128:192]
    i_final = pl.pallas_call(
        _passD_body,
        grid=(n_blk,),
        in_specs=[_row_spec(_BM, _N), _full_spec(_N, w),
                  _row_spec(_BM, _D), _row_spec(_BM, _D)],
        out_specs=_row_spec(_BM, _D),
        out_shape=jax.ShapeDtypeStruct((_N, _D), f32),
        compiler_params=_PARAMS,
    )(iu_graph, out_c, ig0, i1)

    return u_final, i_final
